# baseline (speedup 1.0000x reference)
import functools

import jax
import jax.numpy as jnp
from jax import lax
from jax.experimental import pallas as pl
from jax.experimental.pallas import tpu as pltpu
from jax.experimental.pallas import tpu_sc as plsc

N = 100000
E = 1600000
IN_C = 24
HID = 256
OUT_C = 2
NUM_GRAPHS = 64

NC = 2
NS = 16
L = 16

EPAD = 1638400
NP = 101120
NPAD_ROWS = 1024
ROWS_PER_TILE_Z = NP // NS
ROWS_PER_TILE_W = N // NS
ZCHUNK = 395


def _make_propagate(num_chunks):
    split_edges = num_chunks == 1
    chunks_per_core = 1 if split_edges else num_chunks // 2
    ept = (EPAD // (NC * NS)) if split_edges else (EPAD // NS)
    W = 512
    NW = ept // W
    table_rows = N * num_chunks
    view_len = (N - 1) * num_chunks + 1

    mesh = plsc.VectorSubcoreMesh(
        core_axis_name="c", subcore_axis_name="s", num_cores=NC,
        num_subcores=NS)

    if split_edges:
        out_type = (jax.ShapeDtypeStruct((N, 1, L), jnp.float32),
                    jax.ShapeDtypeStruct((N, 1, L), jnp.float32))
    else:
        out_type = jax.ShapeDtypeStruct((N, num_chunks, L), jnp.float32)

    @functools.partial(
        pl.kernel,
        out_type=out_type,
        mesh=mesh,
        compiler_params=pltpu.CompilerParams(use_tc_tiling_on_sc=False),
        scratch_types=[
            pltpu.VMEM((2, W), jnp.int32),
            pltpu.VMEM((2, W), jnp.int32),
            pltpu.VMEM((2, W, 1, L), jnp.float32),
            pltpu.VMEM((ZCHUNK, 1, L), jnp.float32),
            pltpu.VMEM_SHARED((NP, 1, L), jnp.float32),
            pltpu.SemaphoreType.DMA,
            pltpu.SemaphoreType.DMA,
        ],
    )
    def propagate(table_hbm, srcf_hbm, dst_hbm, *rest):
        if split_edges:
            out_a, out_b, sidx, didx, rows, zbuf, acc, ssem0, ssem1 = rest
        else:
            out_z, sidx, didx, rows, zbuf, acc, ssem0, ssem1 = rest
        ssem = (ssem0, ssem1)
        c = lax.axis_index("c")
        s = lax.axis_index("s")

        def fill_zero(i, _):
            zbuf[i, 0, :] = jnp.zeros((L,), jnp.float32)
            return ()
        lax.fori_loop(0, ZCHUNK, fill_zero, ())

        if split_edges:
            ebase = c * (EPAD // NC) + s * ept
        else:
            ebase = s * ept

        for k in range(chunks_per_core):
            if split_edges:
                kg = jnp.int32(0)
            else:
                kg = c * chunks_per_core + k

            z0 = s * ROWS_PER_TILE_Z
            for zz in range(ROWS_PER_TILE_Z // ZCHUNK):
                pltpu.sync_copy(zbuf, acc.at[pl.ds(z0 + zz * ZCHUNK, ZCHUNK)])
            plsc.subcore_barrier()

            gview = table_hbm.at[pl.ds(kg * 1, view_len)]

            def pair(pp, _):
                for b in (0, 1):
                    base = ebase + (2 * pp + b) * W
                    rb = rows.at[b]
                    db = didx.at[b]

                    @pl.when(pp > 0)
                    def _():
                        pltpu.make_async_copy(
                            rb, acc.at[db], ssem[b]).wait()
                    pltpu.sync_copy(srcf_hbm.at[pl.ds(base, W)], sidx.at[b])
                    pltpu.sync_copy(dst_hbm.at[pl.ds(base, W)], db)
                    pltpu.sync_copy(gview.at[sidx.at[b]], rb)
                    pltpu.async_copy(rb, acc.at[db], ssem[b], add=True)
                return ()
            lax.fori_loop(0, NW // 2, pair, ())
            for b in (0, 1):
                pltpu.make_async_copy(
                    rows.at[b], acc.at[didx.at[b]], ssem[b]).wait()
            plsc.subcore_barrier()

            r0 = s * ROWS_PER_TILE_W
            src_slab = acc.at[pl.ds(r0, ROWS_PER_TILE_W)]
            if split_edges:
                @pl.when(c == 0)
                def _():
                    pltpu.sync_copy(src_slab, out_a.at[pl.ds(r0, ROWS_PER_TILE_W)])
                @pl.when(c == 1)
                def _():
                    pltpu.sync_copy(src_slab, out_b.at[pl.ds(r0, ROWS_PER_TILE_W)])
            else:
                pltpu.sync_copy(
                    src_slab,
                    out_z.at[pl.ds(r0, ROWS_PER_TILE_W), pl.ds(kg, 1)])

    return propagate


_prop16 = _make_propagate(1)
_prop32 = _make_propagate(2)
_prop256 = _make_propagate(16)


_B = 1000


def _tc_prep(qa, qb, xpad):
    def body(qa_ref, qb_ref, x_ref, dis_ref, g0_ref):
        deg = 1.0 + qa_ref[:, 0:1] + qb_ref[:, 0:1]
        dis = lax.rsqrt(deg)
        dis_ref[...] = dis
        g0_ref[...] = dis * x_ref[...]

    return pl.pallas_call(
        body,
        grid=(N // _B,),
        in_specs=[
            pl.BlockSpec((_B, L), lambda i: (i, 0)),
            pl.BlockSpec((_B, L), lambda i: (i, 0)),
            pl.BlockSpec((_B, 2 * L), lambda i: (i, 0)),
        ],
        out_specs=[
            pl.BlockSpec((_B, 1), lambda i: (i, 0)),
            pl.BlockSpec((_B, 2 * L), lambda i: (i, 0)),
        ],
        out_shape=[
            jax.ShapeDtypeStruct((N, 1), jnp.float32),
            jax.ShapeDtypeStruct((N, 2 * L), jnp.float32),
        ],
    )(qa, qb, xpad)


def _tc_layer(z, g, dis, Wm, b, fin):
    def body(z_ref, g_ref, dis_ref, w_ref, b_ref, out_ref):
        dis = dis_ref[...]
        u = dis * (z_ref[...] + g_ref[...])
        h = jnp.dot(u, w_ref[...], preferred_element_type=jnp.float32)
        h = jnp.maximum(h + b_ref[...], 0.0)
        out_ref[...] = dis * h

    return pl.pallas_call(
        body,
        grid=(N // _B,),
        in_specs=[
            pl.BlockSpec((_B, fin), lambda i: (i, 0)),
            pl.BlockSpec((_B, fin), lambda i: (i, 0)),
            pl.BlockSpec((_B, 1), lambda i: (i, 0)),
            pl.BlockSpec((fin, HID), lambda i: (0, 0)),
            pl.BlockSpec((1, HID), lambda i: (0, 0)),
        ],
        out_specs=pl.BlockSpec((_B, HID), lambda i: (i, 0)),
        out_shape=jax.ShapeDtypeStruct((N, HID), jnp.float32),
    )(z, g, dis, Wm, b)


def _tc_layer5(z, g, dis, W5m, b5, W6m, linWp):
    def body(z_ref, g_ref, dis_ref, w5_ref, b5_ref, w6_ref, lwp_ref, out_ref):
        dis = dis_ref[...]
        u = dis * (z_ref[...] + g_ref[...])
        h = jnp.dot(u, w5_ref[...], preferred_element_type=jnp.float32)
        h = jnp.maximum(h + b5_ref[...], 0.0)
        wf = jnp.dot(w6_ref[...], lwp_ref[...],
                     preferred_element_type=jnp.float32)
        out_ref[...] = dis * jnp.dot(h, wf, preferred_element_type=jnp.float32)

    return pl.pallas_call(
        body,
        grid=(N // _B,),
        in_specs=[
            pl.BlockSpec((_B, HID), lambda i: (i, 0)),
            pl.BlockSpec((_B, HID), lambda i: (i, 0)),
            pl.BlockSpec((_B, 1), lambda i: (i, 0)),
            pl.BlockSpec((HID, HID), lambda i: (0, 0)),
            pl.BlockSpec((1, HID), lambda i: (0, 0)),
            pl.BlockSpec((HID, HID), lambda i: (0, 0)),
            pl.BlockSpec((HID, L), lambda i: (0, 0)),
        ],
        out_specs=pl.BlockSpec((_B, L), lambda i: (i, 0)),
        out_shape=jax.ShapeDtypeStruct((N, L), jnp.float32),
    )(z, g, dis, W5m, b5, W6m, linWp)


def _tc_pool(q, qa, qb, dis, batch2d, b6, linWp, linbp):
    nsteps = N // _B

    def body(q_ref, qa_ref, qb_ref, dis_ref, batch_ref, b6_ref, lwp_ref,
             lbp_ref, out_ref, acc_ref):
        i = pl.program_id(0)

        @pl.when(i == 0)
        def _():
            acc_ref[...] = jnp.zeros_like(acc_ref)

        v = dis_ref[...] * (q_ref[...] + qa_ref[...] + qb_ref[...])
        lane = lax.broadcasted_iota(jnp.int32, (_B, L), 1)
        vv = jnp.where(lane == 2, 1.0, v)
        gid = lax.broadcasted_iota(jnp.int32, (_B, NUM_GRAPHS), 1)
        oneh = (batch_ref[...] == gid).astype(jnp.float32)
        part = lax.dot_general(
            oneh, vv, dimension_numbers=(((0,), (0,)), ((), ())),
            preferred_element_type=jnp.float32)
        acc_ref[...] += part

        @pl.when(i == nsteps - 1)
        def _():
            acc = acc_ref[...]
            counts = jnp.maximum(acc[:, 2:3], 1.0)
            bf = jnp.dot(b6_ref[...], lwp_ref[...],
                         preferred_element_type=jnp.float32) + lbp_ref[...]
            logits16 = acc / counts + bf
            l2 = logits16[:, 0:2]
            m = jnp.max(l2, axis=1, keepdims=True)
            lse = m + jnp.log(jnp.sum(jnp.exp(l2 - m), axis=1, keepdims=True))
            out_ref[...] = l2 - lse

    return pl.pallas_call(
        body,
        grid=(nsteps,),
        in_specs=[
            pl.BlockSpec((_B, L), lambda i: (i, 0)),
            pl.BlockSpec((_B, L), lambda i: (i, 0)),
            pl.BlockSpec((_B, L), lambda i: (i, 0)),
            pl.BlockSpec((_B, 1), lambda i: (i, 0)),
            pl.BlockSpec((_B, 1), lambda i: (i, 0)),
            pl.BlockSpec((1, HID), lambda i: (0, 0)),
            pl.BlockSpec((HID, L), lambda i: (0, 0)),
            pl.BlockSpec((1, L), lambda i: (0, 0)),
        ],
        out_specs=pl.BlockSpec((NUM_GRAPHS, OUT_C), lambda i: (0, 0)),
        out_shape=jax.ShapeDtypeStruct((NUM_GRAPHS, OUT_C), jnp.float32),
        scratch_shapes=[pltpu.VMEM((NUM_GRAPHS, L), jnp.float32)],
    )(q, qa, qb, dis, batch2d, b6, linWp, linbp)


def kernel(x, edge_index, batch, W1, b1, W2, b2, W3, b3, W4, b4, W5, b5,
           W6, b6, lin_W, lin_b):
    src = edge_index[0]
    dst = edge_index[1]
    pad = jnp.arange(EPAD - E, dtype=jnp.int32) & (NPAD_ROWS - 1)
    src_p = jnp.concatenate([src, pad])
    dst_p = jnp.concatenate([dst, N + pad])
    src2 = src_p * 2
    src16 = src_p * 16

    xpad = jnp.pad(x, ((0, 0), (0, 2 * L - IN_C)))
    W1p = jnp.pad(W1, ((0, 2 * L - IN_C), (0, 0)))
    linWp = jnp.pad(lin_W, ((0, 0), (0, L - OUT_C)))
    linbp = jnp.pad(lin_b, (0, L - OUT_C)).reshape(1, L)
    ones_t = jnp.ones((N, 1, L), jnp.float32)

    da, db = _prop16(ones_t, src_p, dst_p)
    dis, g0 = _tc_prep(da.reshape(N, L), db.reshape(N, L), xpad)

    z0 = _prop32(g0.reshape(N * 2, 1, L), src2, dst_p).reshape(N, 2 * L)
    g = _tc_layer(z0, g0, dis, W1p, b1.reshape(1, HID), 2 * L)

    for (Wm, b) in ((W2, b2), (W3, b3), (W4, b4)):
        z = _prop256(g.reshape(N * 16, 1, L), src16, dst_p).reshape(N, HID)
        g = _tc_layer(z, g, dis, Wm, b.reshape(1, HID), HID)

    z4 = _prop256(g.reshape(N * 16, 1, L), src16, dst_p).reshape(N, HID)
    q = _tc_layer5(z4, g, dis, W5, b5.reshape(1, HID), W6, linWp)

    qa, qb = _prop16(q.reshape(N, 1, L), src_p, dst_p)
    return _tc_pool(q, qa.reshape(N, L), qb.reshape(N, L), dis,
                    batch.reshape(N, 1), b6.reshape(1, HID), linWp, linbp)


# speedup vs baseline: 1.0610x; 1.0610x over previous
import functools

import jax
import jax.numpy as jnp
from jax import lax
from jax.experimental import pallas as pl
from jax.experimental.pallas import tpu as pltpu
from jax.experimental.pallas import tpu_sc as plsc

N = 100000
E = 1600000
IN_C = 24
HID = 256
OUT_C = 2
NUM_GRAPHS = 64

NC = 2
NS = 16
L = 16

EPAD = 1638400
NP = 101120
NPAD_ROWS = 1024
ROWS_PER_TILE_Z = NP // NS
ROWS_PER_TILE_W = N // NS
ZCHUNK = 395


def _make_propagate(num_chunks):
    split_edges = num_chunks == 1
    chunks_per_core = 1 if split_edges else num_chunks // 2
    ept = (EPAD // (NC * NS)) if split_edges else (EPAD // NS)
    W = 1024
    NW = ept // W
    table_rows = N * num_chunks
    view_len = (N - 1) * num_chunks + 1

    mesh = plsc.VectorSubcoreMesh(
        core_axis_name="c", subcore_axis_name="s", num_cores=NC,
        num_subcores=NS)

    if split_edges:
        out_type = (jax.ShapeDtypeStruct((N, 1, L), jnp.float32),
                    jax.ShapeDtypeStruct((N, 1, L), jnp.float32))
    else:
        out_type = jax.ShapeDtypeStruct((N, num_chunks, L), jnp.float32)

    @functools.partial(
        pl.kernel,
        out_type=out_type,
        mesh=mesh,
        compiler_params=pltpu.CompilerParams(use_tc_tiling_on_sc=False),
        scratch_types=[
            pltpu.VMEM((W,), jnp.int32),
            pltpu.VMEM((W,), jnp.int32),
            pltpu.VMEM((W, 1, L), jnp.float32),
            pltpu.VMEM((ZCHUNK, 1, L), jnp.float32),
            pltpu.VMEM_SHARED((NP, 1, L), jnp.float32),
        ],
    )
    def propagate(table_hbm, srcf_hbm, dst_hbm, *rest):
        if split_edges:
            out_a, out_b, sidx, didx, rows, zbuf, acc = rest
        else:
            out_z, sidx, didx, rows, zbuf, acc = rest
        c = lax.axis_index("c")
        s = lax.axis_index("s")

        def fill_zero(i, _):
            zbuf[i, 0, :] = jnp.zeros((L,), jnp.float32)
            return ()
        lax.fori_loop(0, ZCHUNK, fill_zero, ())

        if split_edges:
            ebase = c * (EPAD // NC) + s * ept
        else:
            ebase = s * ept

        for k in range(chunks_per_core):
            if split_edges:
                kg = jnp.int32(0)
            else:
                kg = c * chunks_per_core + k

            z0 = s * ROWS_PER_TILE_Z
            for zz in range(ROWS_PER_TILE_Z // ZCHUNK):
                pltpu.sync_copy(zbuf, acc.at[pl.ds(z0 + zz * ZCHUNK, ZCHUNK)])
            plsc.subcore_barrier()

            gview = table_hbm.at[pl.ds(kg * 1, view_len)]

            def window(w, _):
                base = ebase + w * W
                pltpu.sync_copy(srcf_hbm.at[pl.ds(base, W)], sidx)
                pltpu.sync_copy(dst_hbm.at[pl.ds(base, W)], didx)
                pltpu.sync_copy(gview.at[sidx], rows)
                pltpu.sync_copy(rows, acc.at[didx], add=True)
                return ()
            lax.fori_loop(0, NW, window, ())
            plsc.subcore_barrier()

            r0 = s * ROWS_PER_TILE_W
            src_slab = acc.at[pl.ds(r0, ROWS_PER_TILE_W)]
            if split_edges:
                @pl.when(c == 0)
                def _():
                    pltpu.sync_copy(src_slab, out_a.at[pl.ds(r0, ROWS_PER_TILE_W)])
                @pl.when(c == 1)
                def _():
                    pltpu.sync_copy(src_slab, out_b.at[pl.ds(r0, ROWS_PER_TILE_W)])
            else:
                pltpu.sync_copy(
                    src_slab,
                    out_z.at[pl.ds(r0, ROWS_PER_TILE_W), pl.ds(kg, 1)])

    return propagate


_prop16 = _make_propagate(1)
_prop32 = _make_propagate(2)
_prop256 = _make_propagate(16)


_B = 1000


def _tc_prep(qa, qb, xpad):
    def body(qa_ref, qb_ref, x_ref, dis_ref, g0_ref):
        deg = 1.0 + qa_ref[:, 0:1] + qb_ref[:, 0:1]
        dis = lax.rsqrt(deg)
        dis_ref[...] = dis
        g0_ref[...] = dis * x_ref[...]

    return pl.pallas_call(
        body,
        grid=(N // _B,),
        in_specs=[
            pl.BlockSpec((_B, L), lambda i: (i, 0)),
            pl.BlockSpec((_B, L), lambda i: (i, 0)),
            pl.BlockSpec((_B, 2 * L), lambda i: (i, 0)),
        ],
        out_specs=[
            pl.BlockSpec((_B, 1), lambda i: (i, 0)),
            pl.BlockSpec((_B, 2 * L), lambda i: (i, 0)),
        ],
        out_shape=[
            jax.ShapeDtypeStruct((N, 1), jnp.float32),
            jax.ShapeDtypeStruct((N, 2 * L), jnp.float32),
        ],
    )(qa, qb, xpad)


def _tc_layer(z, g, dis, Wm, b, fin):
    def body(z_ref, g_ref, dis_ref, w_ref, b_ref, out_ref):
        dis = dis_ref[...]
        u = dis * (z_ref[...] + g_ref[...])
        h = jnp.dot(u, w_ref[...], preferred_element_type=jnp.float32)
        h = jnp.maximum(h + b_ref[...], 0.0)
        out_ref[...] = dis * h

    return pl.pallas_call(
        body,
        grid=(N // _B,),
        in_specs=[
            pl.BlockSpec((_B, fin), lambda i: (i, 0)),
            pl.BlockSpec((_B, fin), lambda i: (i, 0)),
            pl.BlockSpec((_B, 1), lambda i: (i, 0)),
            pl.BlockSpec((fin, HID), lambda i: (0, 0)),
            pl.BlockSpec((1, HID), lambda i: (0, 0)),
        ],
        out_specs=pl.BlockSpec((_B, HID), lambda i: (i, 0)),
        out_shape=jax.ShapeDtypeStruct((N, HID), jnp.float32),
    )(z, g, dis, Wm, b)


def _tc_layer5(z, g, dis, W5m, b5, W6m, linWp):
    def body(z_ref, g_ref, dis_ref, w5_ref, b5_ref, w6_ref, lwp_ref, out_ref):
        dis = dis_ref[...]
        u = dis * (z_ref[...] + g_ref[...])
        h = jnp.dot(u, w5_ref[...], preferred_element_type=jnp.float32)
        h = jnp.maximum(h + b5_ref[...], 0.0)
        wf = jnp.dot(w6_ref[...], lwp_ref[...],
                     preferred_element_type=jnp.float32)
        out_ref[...] = dis * jnp.dot(h, wf, preferred_element_type=jnp.float32)

    return pl.pallas_call(
        body,
        grid=(N // _B,),
        in_specs=[
            pl.BlockSpec((_B, HID), lambda i: (i, 0)),
            pl.BlockSpec((_B, HID), lambda i: (i, 0)),
            pl.BlockSpec((_B, 1), lambda i: (i, 0)),
            pl.BlockSpec((HID, HID), lambda i: (0, 0)),
            pl.BlockSpec((1, HID), lambda i: (0, 0)),
            pl.BlockSpec((HID, HID), lambda i: (0, 0)),
            pl.BlockSpec((HID, L), lambda i: (0, 0)),
        ],
        out_specs=pl.BlockSpec((_B, L), lambda i: (i, 0)),
        out_shape=jax.ShapeDtypeStruct((N, L), jnp.float32),
    )(z, g, dis, W5m, b5, W6m, linWp)


def _tc_pool(q, qa, qb, dis, batch2d, b6, linWp, linbp):
    nsteps = N // _B

    def body(q_ref, qa_ref, qb_ref, dis_ref, batch_ref, b6_ref, lwp_ref,
             lbp_ref, out_ref, acc_ref):
        i = pl.program_id(0)

        @pl.when(i == 0)
        def _():
            acc_ref[...] = jnp.zeros_like(acc_ref)

        v = dis_ref[...] * (q_ref[...] + qa_ref[...] + qb_ref[...])
        lane = lax.broadcasted_iota(jnp.int32, (_B, L), 1)
        vv = jnp.where(lane == 2, 1.0, v)
        gid = lax.broadcasted_iota(jnp.int32, (_B, NUM_GRAPHS), 1)
        oneh = (batch_ref[...] == gid).astype(jnp.float32)
        part = lax.dot_general(
            oneh, vv, dimension_numbers=(((0,), (0,)), ((), ())),
            preferred_element_type=jnp.float32)
        acc_ref[...] += part

        @pl.when(i == nsteps - 1)
        def _():
            acc = acc_ref[...]
            counts = jnp.maximum(acc[:, 2:3], 1.0)
            bf = jnp.dot(b6_ref[...], lwp_ref[...],
                         preferred_element_type=jnp.float32) + lbp_ref[...]
            logits16 = acc / counts + bf
            l2 = logits16[:, 0:2]
            m = jnp.max(l2, axis=1, keepdims=True)
            lse = m + jnp.log(jnp.sum(jnp.exp(l2 - m), axis=1, keepdims=True))
            out_ref[...] = l2 - lse

    return pl.pallas_call(
        body,
        grid=(nsteps,),
        in_specs=[
            pl.BlockSpec((_B, L), lambda i: (i, 0)),
            pl.BlockSpec((_B, L), lambda i: (i, 0)),
            pl.BlockSpec((_B, L), lambda i: (i, 0)),
            pl.BlockSpec((_B, 1), lambda i: (i, 0)),
            pl.BlockSpec((_B, 1), lambda i: (i, 0)),
            pl.BlockSpec((1, HID), lambda i: (0, 0)),
            pl.BlockSpec((HID, L), lambda i: (0, 0)),
            pl.BlockSpec((1, L), lambda i: (0, 0)),
        ],
        out_specs=pl.BlockSpec((NUM_GRAPHS, OUT_C), lambda i: (0, 0)),
        out_shape=jax.ShapeDtypeStruct((NUM_GRAPHS, OUT_C), jnp.float32),
        scratch_shapes=[pltpu.VMEM((NUM_GRAPHS, L), jnp.float32)],
    )(q, qa, qb, dis, batch2d, b6, linWp, linbp)


def kernel(x, edge_index, batch, W1, b1, W2, b2, W3, b3, W4, b4, W5, b5,
           W6, b6, lin_W, lin_b):
    src = edge_index[0]
    dst = edge_index[1]
    pad = jnp.arange(EPAD - E, dtype=jnp.int32) & (NPAD_ROWS - 1)
    src_p = jnp.concatenate([src, pad])
    dst_p = jnp.concatenate([dst, N + pad])
    src2 = src_p * 2
    src16 = src_p * 16

    xpad = jnp.pad(x, ((0, 0), (0, 2 * L - IN_C)))
    W1p = jnp.pad(W1, ((0, 2 * L - IN_C), (0, 0)))
    linWp = jnp.pad(lin_W, ((0, 0), (0, L - OUT_C)))
    linbp = jnp.pad(lin_b, (0, L - OUT_C)).reshape(1, L)
    ones_t = jnp.ones((N, 1, L), jnp.float32)

    da, db = _prop16(ones_t, src_p, dst_p)
    dis, g0 = _tc_prep(da.reshape(N, L), db.reshape(N, L), xpad)

    z0 = _prop32(g0.reshape(N * 2, 1, L), src2, dst_p).reshape(N, 2 * L)
    g = _tc_layer(z0, g0, dis, W1p, b1.reshape(1, HID), 2 * L)

    for (Wm, b) in ((W2, b2), (W3, b3), (W4, b4)):
        z = _prop256(g.reshape(N * 16, 1, L), src16, dst_p).reshape(N, HID)
        g = _tc_layer(z, g, dis, Wm, b.reshape(1, HID), HID)

    z4 = _prop256(g.reshape(N * 16, 1, L), src16, dst_p).reshape(N, HID)
    q = _tc_layer5(z4, g, dis, W5, b5.reshape(1, HID), W6, linWp)

    qa, qb = _prop16(q.reshape(N, 1, L), src_p, dst_p)
    return _tc_pool(q, qa.reshape(N, L), qb.reshape(N, L), dis,
                    batch.reshape(N, 1), b6.reshape(1, HID), linWp, linbp)


# speedup vs baseline: 1.0838x; 1.0215x over previous
import functools

import jax
import jax.numpy as jnp
from jax import lax
from jax.experimental import pallas as pl
from jax.experimental.pallas import tpu as pltpu
from jax.experimental.pallas import tpu_sc as plsc

N = 100000
E = 1600000
IN_C = 24
HID = 256
OUT_C = 2
NUM_GRAPHS = 64

NC = 2
NS = 16
L = 16

EPAD = 1638400
NP = 101120
NPAD_ROWS = 1024
ROWS_PER_TILE_Z = NP // NS
ROWS_PER_TILE_W = N // NS
ZCHUNK = 395


def _make_propagate(num_chunks):
    split_edges = num_chunks == 1
    chunks_per_core = 1 if split_edges else num_chunks // 2
    ept = (EPAD // (NC * NS)) if split_edges else (EPAD // NS)
    W = 1280
    NW = ept // W
    table_rows = N * num_chunks
    view_len = (N - 1) * num_chunks + 1

    mesh = plsc.VectorSubcoreMesh(
        core_axis_name="c", subcore_axis_name="s", num_cores=NC,
        num_subcores=NS)

    if split_edges:
        out_type = (jax.ShapeDtypeStruct((N, 1, L), jnp.float32),
                    jax.ShapeDtypeStruct((N, 1, L), jnp.float32))
    else:
        out_type = jax.ShapeDtypeStruct((N, num_chunks, L), jnp.float32)

    @functools.partial(
        pl.kernel,
        out_type=out_type,
        mesh=mesh,
        compiler_params=pltpu.CompilerParams(use_tc_tiling_on_sc=False),
        scratch_types=[
            pltpu.VMEM((W,), jnp.int32),
            pltpu.VMEM((W,), jnp.int32),
            pltpu.VMEM((W, 1, L), jnp.float32),
            pltpu.VMEM((ZCHUNK, 1, L), jnp.float32),
            pltpu.VMEM_SHARED((NP, 1, L), jnp.float32),
        ],
    )
    def propagate(table_hbm, srcf_hbm, dst_hbm, *rest):
        if split_edges:
            out_a, out_b, sidx, didx, rows, zbuf, acc = rest
        else:
            out_z, sidx, didx, rows, zbuf, acc = rest
        c = lax.axis_index("c")
        s = lax.axis_index("s")

        def fill_zero(i, _):
            zbuf[i, 0, :] = jnp.zeros((L,), jnp.float32)
            return ()
        lax.fori_loop(0, ZCHUNK, fill_zero, ())

        if split_edges:
            ebase = c * (EPAD // NC) + s * ept
        else:
            ebase = s * ept

        for k in range(chunks_per_core):
            if split_edges:
                kg = jnp.int32(0)
            else:
                kg = c * chunks_per_core + k

            z0 = s * ROWS_PER_TILE_Z
            for zz in range(ROWS_PER_TILE_Z // ZCHUNK):
                pltpu.sync_copy(zbuf, acc.at[pl.ds(z0 + zz * ZCHUNK, ZCHUNK)])
            plsc.subcore_barrier()

            gview = table_hbm.at[pl.ds(kg * 1, view_len)]

            def window(w, _):
                base = ebase + w * W
                pltpu.sync_copy(srcf_hbm.at[pl.ds(base, W)], sidx)
                pltpu.sync_copy(dst_hbm.at[pl.ds(base, W)], didx)
                pltpu.sync_copy(gview.at[sidx], rows)
                pltpu.sync_copy(rows, acc.at[didx], add=True)
                return ()
            lax.fori_loop(0, NW, window, ())
            plsc.subcore_barrier()

            r0 = s * ROWS_PER_TILE_W
            src_slab = acc.at[pl.ds(r0, ROWS_PER_TILE_W)]
            if split_edges:
                @pl.when(c == 0)
                def _():
                    pltpu.sync_copy(src_slab, out_a.at[pl.ds(r0, ROWS_PER_TILE_W)])
                @pl.when(c == 1)
                def _():
                    pltpu.sync_copy(src_slab, out_b.at[pl.ds(r0, ROWS_PER_TILE_W)])
            else:
                pltpu.sync_copy(
                    src_slab,
                    out_z.at[pl.ds(r0, ROWS_PER_TILE_W), pl.ds(kg, 1)])

    return propagate


_prop16 = _make_propagate(1)
_prop32 = _make_propagate(2)
_prop256 = _make_propagate(16)


_B = 1000


def _tc_prep(qa, qb, xpad):
    def body(qa_ref, qb_ref, x_ref, dis_ref, g0_ref):
        deg = 1.0 + qa_ref[:, 0:1] + qb_ref[:, 0:1]
        dis = lax.rsqrt(deg)
        dis_ref[...] = dis
        g0_ref[...] = dis * x_ref[...]

    return pl.pallas_call(
        body,
        grid=(N // _B,),
        in_specs=[
            pl.BlockSpec((_B, L), lambda i: (i, 0)),
            pl.BlockSpec((_B, L), lambda i: (i, 0)),
            pl.BlockSpec((_B, 2 * L), lambda i: (i, 0)),
        ],
        out_specs=[
            pl.BlockSpec((_B, 1), lambda i: (i, 0)),
            pl.BlockSpec((_B, 2 * L), lambda i: (i, 0)),
        ],
        out_shape=[
            jax.ShapeDtypeStruct((N, 1), jnp.float32),
            jax.ShapeDtypeStruct((N, 2 * L), jnp.float32),
        ],
    )(qa, qb, xpad)


def _tc_layer(z, g, dis, Wm, b, fin):
    def body(z_ref, g_ref, dis_ref, w_ref, b_ref, out_ref):
        dis = dis_ref[...]
        u = dis * (z_ref[...] + g_ref[...])
        h = jnp.dot(u, w_ref[...], preferred_element_type=jnp.float32)
        h = jnp.maximum(h + b_ref[...], 0.0)
        out_ref[...] = dis * h

    return pl.pallas_call(
        body,
        grid=(N // _B,),
        in_specs=[
            pl.BlockSpec((_B, fin), lambda i: (i, 0)),
            pl.BlockSpec((_B, fin), lambda i: (i, 0)),
            pl.BlockSpec((_B, 1), lambda i: (i, 0)),
            pl.BlockSpec((fin, HID), lambda i: (0, 0)),
            pl.BlockSpec((1, HID), lambda i: (0, 0)),
        ],
        out_specs=pl.BlockSpec((_B, HID), lambda i: (i, 0)),
        out_shape=jax.ShapeDtypeStruct((N, HID), jnp.float32),
    )(z, g, dis, Wm, b)


def _tc_layer5(z, g, dis, W5m, b5, W6m, linWp):
    def body(z_ref, g_ref, dis_ref, w5_ref, b5_ref, w6_ref, lwp_ref, out_ref):
        dis = dis_ref[...]
        u = dis * (z_ref[...] + g_ref[...])
        h = jnp.dot(u, w5_ref[...], preferred_element_type=jnp.float32)
        h = jnp.maximum(h + b5_ref[...], 0.0)
        wf = jnp.dot(w6_ref[...], lwp_ref[...],
                     preferred_element_type=jnp.float32)
        out_ref[...] = dis * jnp.dot(h, wf, preferred_element_type=jnp.float32)

    return pl.pallas_call(
        body,
        grid=(N // _B,),
        in_specs=[
            pl.BlockSpec((_B, HID), lambda i: (i, 0)),
            pl.BlockSpec((_B, HID), lambda i: (i, 0)),
            pl.BlockSpec((_B, 1), lambda i: (i, 0)),
            pl.BlockSpec((HID, HID), lambda i: (0, 0)),
            pl.BlockSpec((1, HID), lambda i: (0, 0)),
            pl.BlockSpec((HID, HID), lambda i: (0, 0)),
            pl.BlockSpec((HID, L), lambda i: (0, 0)),
        ],
        out_specs=pl.BlockSpec((_B, L), lambda i: (i, 0)),
        out_shape=jax.ShapeDtypeStruct((N, L), jnp.float32),
    )(z, g, dis, W5m, b5, W6m, linWp)


def _tc_pool(q, qa, qb, dis, batch2d, b6, linWp, linbp):
    nsteps = N // _B

    def body(q_ref, qa_ref, qb_ref, dis_ref, batch_ref, b6_ref, lwp_ref,
             lbp_ref, out_ref, acc_ref):
        i = pl.program_id(0)

        @pl.when(i == 0)
        def _():
            acc_ref[...] = jnp.zeros_like(acc_ref)

        v = dis_ref[...] * (q_ref[...] + qa_ref[...] + qb_ref[...])
        lane = lax.broadcasted_iota(jnp.int32, (_B, L), 1)
        vv = jnp.where(lane == 2, 1.0, v)
        gid = lax.broadcasted_iota(jnp.int32, (_B, NUM_GRAPHS), 1)
        oneh = (batch_ref[...] == gid).astype(jnp.float32)
        part = lax.dot_general(
            oneh, vv, dimension_numbers=(((0,), (0,)), ((), ())),
            preferred_element_type=jnp.float32)
        acc_ref[...] += part

        @pl.when(i == nsteps - 1)
        def _():
            acc = acc_ref[...]
            counts = jnp.maximum(acc[:, 2:3], 1.0)
            bf = jnp.dot(b6_ref[...], lwp_ref[...],
                         preferred_element_type=jnp.float32) + lbp_ref[...]
            logits16 = acc / counts + bf
            l2 = logits16[:, 0:2]
            m = jnp.max(l2, axis=1, keepdims=True)
            lse = m + jnp.log(jnp.sum(jnp.exp(l2 - m), axis=1, keepdims=True))
            out_ref[...] = l2 - lse

    return pl.pallas_call(
        body,
        grid=(nsteps,),
        in_specs=[
            pl.BlockSpec((_B, L), lambda i: (i, 0)),
            pl.BlockSpec((_B, L), lambda i: (i, 0)),
            pl.BlockSpec((_B, L), lambda i: (i, 0)),
            pl.BlockSpec((_B, 1), lambda i: (i, 0)),
            pl.BlockSpec((_B, 1), lambda i: (i, 0)),
            pl.BlockSpec((1, HID), lambda i: (0, 0)),
            pl.BlockSpec((HID, L), lambda i: (0, 0)),
            pl.BlockSpec((1, L), lambda i: (0, 0)),
        ],
        out_specs=pl.BlockSpec((NUM_GRAPHS, OUT_C), lambda i: (0, 0)),
        out_shape=jax.ShapeDtypeStruct((NUM_GRAPHS, OUT_C), jnp.float32),
        scratch_shapes=[pltpu.VMEM((NUM_GRAPHS, L), jnp.float32)],
    )(q, qa, qb, dis, batch2d, b6, linWp, linbp)


def kernel(x, edge_index, batch, W1, b1, W2, b2, W3, b3, W4, b4, W5, b5,
           W6, b6, lin_W, lin_b):
    src = edge_index[0]
    dst = edge_index[1]
    pad = jnp.arange(EPAD - E, dtype=jnp.int32) & (NPAD_ROWS - 1)
    src_p = jnp.concatenate([src, pad])
    dst_p = jnp.concatenate([dst, N + pad])
    src2 = src_p * 2
    src16 = src_p * 16

    xpad = jnp.pad(x, ((0, 0), (0, 2 * L - IN_C)))
    W1p = jnp.pad(W1, ((0, 2 * L - IN_C), (0, 0)))
    linWp = jnp.pad(lin_W, ((0, 0), (0, L - OUT_C)))
    linbp = jnp.pad(lin_b, (0, L - OUT_C)).reshape(1, L)
    ones_t = jnp.ones((N, 1, L), jnp.float32)

    da, db = _prop16(ones_t, src_p, dst_p)
    dis, g0 = _tc_prep(da.reshape(N, L), db.reshape(N, L), xpad)

    z0 = _prop32(g0.reshape(N * 2, 1, L), src2, dst_p).reshape(N, 2 * L)
    g = _tc_layer(z0, g0, dis, W1p, b1.reshape(1, HID), 2 * L)

    for (Wm, b) in ((W2, b2), (W3, b3), (W4, b4)):
        z = _prop256(g.reshape(N * 16, 1, L), src16, dst_p).reshape(N, HID)
        g = _tc_layer(z, g, dis, Wm, b.reshape(1, HID), HID)

    z4 = _prop256(g.reshape(N * 16, 1, L), src16, dst_p).reshape(N, HID)
    q = _tc_layer5(z4, g, dis, W5, b5.reshape(1, HID), W6, linWp)

    qa, qb = _prop16(q.reshape(N, 1, L), src_p, dst_p)
    return _tc_pool(q, qa.reshape(N, L), qb.reshape(N, L), dis,
                    batch.reshape(N, 1), b6.reshape(1, HID), linWp, linbp)
